# Initial kernel scaffold; baseline (speedup 1.0000x reference)
import functools

import jax
import jax.numpy as jnp
from jax import lax
from jax.experimental import pallas as pl
from jax.experimental.pallas import tpu as pltpu
from jax.experimental.pallas import tpu_sc as plsc

_CROSS_L = 3
_EPS = 1e-5
_BLK = 1024


def _make_sc_gather(rows, emb_dim, rows_per_w, chunk, num_cores):
    mesh = plsc.VectorSubcoreMesh(core_axis_name="c", subcore_axis_name="s")

    @functools.partial(
        pl.kernel,
        out_type=jax.ShapeDtypeStruct((rows, emb_dim), jnp.float32),
        mesh=mesh,
        scratch_types=[
            pltpu.VMEM((chunk,), jnp.int32),
            pltpu.VMEM((chunk, emb_dim), jnp.float32),
            pltpu.SemaphoreType.DMA,
        ],
    )
    def gather_k(table_hbm, gidx_hbm, out_hbm, idx_v, rows_v, sem):
        wid = lax.axis_index("s") * num_cores + lax.axis_index("c")
        base = wid * rows_per_w

        def body(c, carry):
            off = base + c * chunk
            pltpu.sync_copy(gidx_hbm.at[pl.ds(off, chunk)], idx_v)
            pltpu.async_copy(table_hbm.at[idx_v], rows_v, sem).wait()
            pltpu.sync_copy(rows_v, out_hbm.at[pl.ds(off, chunk)])
            return carry

        lax.fori_loop(0, rows_per_w // chunk, body, 0)

    return gather_k


def _pass_a(emb_ref, den_ref, wct_ref, stats_e_ref, stats_d_ref, st_ref, *, de):
    i = pl.program_id(0)
    emb = emb_ref[...]
    den = den_ref[...]
    st = lax.dot_general(emb, wct_ref[0:de, :], (((1,), (0,)), ((), ())),
                         precision=lax.Precision.HIGHEST,
                         preferred_element_type=jnp.float32)
    st += lax.dot_general(den, wct_ref[de:, :], (((1,), (0,)), ((), ())),
                          precision=lax.Precision.HIGHEST,
                          preferred_element_type=jnp.float32)
    st_ref[...] = st

    @pl.when(i == 0)
    def _():
        stats_e_ref[...] = jnp.zeros_like(stats_e_ref)
        stats_d_ref[...] = jnp.zeros_like(stats_d_ref)

    stats_e_ref[0:2, :] += jnp.concatenate(
        [jnp.sum(emb, axis=0, keepdims=True),
         jnp.sum(emb * emb, axis=0, keepdims=True)], axis=0)
    stats_d_ref[0:2, :] += jnp.concatenate(
        [jnp.sum(den, axis=0, keepdims=True),
         jnp.sum(den * den, axis=0, keepdims=True)], axis=0)


def _norm(x, stats_ref, g_ref, b_ref, inv_b):
    m = stats_ref[0:1, :] * inv_b
    v = stats_ref[1:2, :] * inv_b - m * m
    a = lax.rsqrt(v + _EPS) * g_ref[...]
    return (x - m) * a + b_ref[...]


def _pass_b(emb_ref, den_ref, stats_e_ref, stats_d_ref, ge_ref, be_ref,
            gd_ref, bd_ref, w1t_ref, b1_ref, h1_ref, stats1_ref, *, bsz, de):
    i = pl.program_id(0)
    inv_b = 1.0 / bsz
    xe = _norm(emb_ref[...], stats_e_ref, ge_ref, be_ref, inv_b)
    xd = _norm(den_ref[...], stats_d_ref, gd_ref, bd_ref, inv_b)
    h1 = lax.dot_general(xe, w1t_ref[0:de, :], (((1,), (0,)), ((), ())),
                         precision=lax.Precision.HIGHEST,
                         preferred_element_type=jnp.float32)
    h1 += lax.dot_general(xd, w1t_ref[de:, :], (((1,), (0,)), ((), ())),
                          precision=lax.Precision.HIGHEST,
                          preferred_element_type=jnp.float32)
    h1 += b1_ref[...]
    h1_ref[...] = h1

    @pl.when(i == 0)
    def _():
        stats1_ref[...] = jnp.zeros_like(stats1_ref)

    stats1_ref[0:2, :] += jnp.concatenate(
        [jnp.sum(h1, axis=0, keepdims=True),
         jnp.sum(h1 * h1, axis=0, keepdims=True)], axis=0)


def _pass_c(h1_ref, stats1_ref, g1_ref, b1n_ref, w2t_ref, b2_ref,
            h2_ref, stats2_ref, *, bsz):
    i = pl.program_id(0)
    xn = _norm(h1_ref[...], stats1_ref, g1_ref, b1n_ref, 1.0 / bsz)
    h2 = lax.dot_general(xn, w2t_ref[...], (((1,), (0,)), ((), ())),
                         precision=lax.Precision.HIGHEST,
                         preferred_element_type=jnp.float32) + b2_ref[...]
    h2_ref[...] = h2

    @pl.when(i == 0)
    def _():
        stats2_ref[...] = jnp.zeros_like(stats2_ref)

    stats2_ref[0:2, :] += jnp.concatenate(
        [jnp.sum(h2, axis=0, keepdims=True),
         jnp.sum(h2 * h2, axis=0, keepdims=True)], axis=0)


def _pass_d(h2_ref, st_ref, stats2_ref, g2_ref, b2n_ref, wpe_ref, wpd_ref,
            cb_ref, bp_ref, out_ref, *, bsz):
    h2n = _norm(h2_ref[...], stats2_ref, g2_ref, b2n_ref, 1.0 / bsz)
    dpart = jnp.sum(h2n * wpd_ref[...], axis=1, keepdims=True)
    wpe_sum = jnp.sum(wpe_ref[...], axis=1, keepdims=True)
    s = st_ref[:, 0:1]
    t = st_ref[:, 1:2]
    cb_last = cb_ref[0:1, _CROSS_L - 1:_CROSS_L]
    logit = (1.0 + s) * t + cb_last * wpe_sum + dpart + bp_ref[0:1, 0:1]
    out_ref[...] = jax.nn.sigmoid(logit)


def kernel(sparse_data, dense_data, emb_tables, cross_w, cross_b, bn0_g, bn0_b,
           w1, b1, bn1_g, bn1_b, w2, b2, bn2_g, bn2_b, wp, bp):
    bsz = sparse_data.shape[0]
    f, vocab, e = emb_tables.shape
    de = f * e
    dd = dense_data.shape[1]
    d_in = de + dd
    h1dim = w1.shape[0]
    h2dim = w2.shape[0]

    idx0 = sparse_data[:, 0].astype(jnp.int32)
    gidx = (idx0[:, None]
            + (jnp.arange(f, dtype=jnp.int32) * vocab)[None, :]).reshape(-1)
    table = emb_tables.reshape(f * vocab, e)

    rows = bsz * f
    info = plsc.get_sparse_core_info()
    nw = info.num_cores * info.num_subcores
    rows_per_w = rows // nw
    chunk = 1664
    assert rows_per_w % chunk == 0 and chunk % 8 == 0
    emb_flat = _make_sc_gather(rows, e, rows_per_w, chunk, info.num_cores)(
        table, gidx)
    emb = emb_flat.reshape(bsz, de)

    nblk = bsz // _BLK
    grid = (nblk,)
    row2 = lambda n: jnp.reshape(n, (1, -1))

    wct = jnp.stack([cross_w[_CROSS_L - 1], wp[0, :d_in]], axis=1)
    w1t = w1.T
    w2t = w2.T

    def bspec(shape_rows, shape_cols, blocked=True):
        if blocked:
            return pl.BlockSpec((shape_rows, shape_cols), lambda i: (i, 0))
        return pl.BlockSpec((shape_rows, shape_cols), lambda i: (0, 0))

    f32 = jnp.float32
    stats_e, stats_d, st = pl.pallas_call(
        functools.partial(_pass_a, de=de),
        grid=grid,
        in_specs=[
            bspec(_BLK, de), bspec(_BLK, dd), bspec(d_in, 2, False),
        ],
        out_specs=[
            bspec(8, de, False), bspec(8, dd, False), bspec(_BLK, 2),
        ],
        out_shape=[
            jax.ShapeDtypeStruct((8, de), f32),
            jax.ShapeDtypeStruct((8, dd), f32),
            jax.ShapeDtypeStruct((bsz, 2), f32),
        ],
    )(emb, dense_data, wct)

    h1, stats1 = pl.pallas_call(
        functools.partial(_pass_b, bsz=bsz, de=de),
        grid=grid,
        in_specs=[
            bspec(_BLK, de), bspec(_BLK, dd),
            bspec(8, de, False), bspec(8, dd, False),
            bspec(1, de, False), bspec(1, de, False),
            bspec(1, dd, False), bspec(1, dd, False),
            bspec(d_in, h1dim, False), bspec(1, h1dim, False),
        ],
        out_specs=[bspec(_BLK, h1dim), bspec(8, h1dim, False)],
        out_shape=[
            jax.ShapeDtypeStruct((bsz, h1dim), f32),
            jax.ShapeDtypeStruct((8, h1dim), f32),
        ],
    )(emb, dense_data, stats_e, stats_d,
      row2(bn0_g[:de]), row2(bn0_b[:de]), row2(bn0_g[de:]), row2(bn0_b[de:]),
      w1t, row2(b1))

    h2, stats2 = pl.pallas_call(
        functools.partial(_pass_c, bsz=bsz),
        grid=grid,
        in_specs=[
            bspec(_BLK, h1dim),
            bspec(8, h1dim, False),
            bspec(1, h1dim, False), bspec(1, h1dim, False),
            bspec(h1dim, h2dim, False), bspec(1, h2dim, False),
        ],
        out_specs=[bspec(_BLK, h2dim), bspec(8, h2dim, False)],
        out_shape=[
            jax.ShapeDtypeStruct((bsz, h2dim), f32),
            jax.ShapeDtypeStruct((8, h2dim), f32),
        ],
    )(h1, stats1, row2(bn1_g), row2(bn1_b), w2t, row2(b2))

    out = pl.pallas_call(
        functools.partial(_pass_d, bsz=bsz),
        grid=grid,
        in_specs=[
            bspec(_BLK, h2dim), bspec(_BLK, 2),
            bspec(8, h2dim, False),
            bspec(1, h2dim, False), bspec(1, h2dim, False),
            bspec(1, d_in, False), bspec(1, h2dim, False),
            bspec(1, _CROSS_L, False), bspec(1, 1, False),
        ],
        out_specs=bspec(_BLK, 1),
        out_shape=jax.ShapeDtypeStruct((bsz, 1), f32),
    )(h2, st, stats2, row2(bn2_g), row2(bn2_b),
      row2(wp[0, :d_in]), row2(wp[0, d_in:]), row2(cross_b), row2(bp))

    return out


# speedup vs baseline: 1.5182x; 1.5182x over previous
import functools

import jax
import jax.numpy as jnp
from jax import lax
from jax.experimental import pallas as pl
from jax.experimental.pallas import tpu as pltpu
from jax.experimental.pallas import tpu_sc as plsc

_CROSS_L = 3
_EPS = 1e-5
_BLK = 1024


_CHUNK = 128


def _make_sc_gather(bsz, f, e, nb_per_w, num_cores):
    mesh = plsc.VectorSubcoreMesh(core_axis_name="c", subcore_axis_name="s")
    nch_per_w = nb_per_w * f

    @functools.partial(
        pl.kernel,
        out_type=jax.ShapeDtypeStruct((bsz, f, e), jnp.float32),
        mesh=mesh,
        compiler_params=pltpu.CompilerParams(use_tc_tiling_on_sc=False),
        scratch_types=[
            pltpu.VMEM((nb_per_w, _CHUNK), jnp.int32),
            pltpu.VMEM((_CHUNK, e), jnp.float32),
            pltpu.VMEM((_CHUNK, e), jnp.float32),
            pltpu.SemaphoreType.DMA,
            pltpu.SemaphoreType.DMA,
        ],
    )
    def gather_k(table_hbm, idx_hbm, out_hbm, idx_v, buf0, buf1, sem0, sem1):
        wid = lax.axis_index("s") * num_cores + lax.axis_index("c")
        base = wid * nb_per_w * _CHUNK
        pltpu.sync_copy(idx_hbm.at[wid], idx_v)

        def src(c):
            r = c // f
            fld = c - r * f
            return table_hbm.at[fld].at[idx_v.at[r]]

        def start(c, buf, sem):
            pltpu.async_copy(src(c), buf, sem)

        def drain(c, buf, sem):
            r = c // f
            fld = c - r * f
            pltpu.make_async_copy(src(c), buf, sem).wait()
            pltpu.sync_copy(
                buf, out_hbm.at[pl.ds(base + r * _CHUNK, _CHUNK), fld])

        start(0, buf0, sem0)

        def body(i, carry):
            c = 2 * i
            start(c + 1, buf1, sem1)
            drain(c, buf0, sem0)

            @pl.when(c + 2 < nch_per_w)
            def _():
                start(c + 2, buf0, sem0)

            drain(c + 1, buf1, sem1)
            return carry

        lax.fori_loop(0, nch_per_w // 2, body, 0)

    return gather_k


def _pass_a(emb_ref, den_ref, wct_ref, stats_e_ref, stats_d_ref, st_ref, *, de):
    i = pl.program_id(0)
    emb = emb_ref[...]
    den = den_ref[...]
    st = lax.dot_general(emb, wct_ref[0:de, :], (((1,), (0,)), ((), ())),
                         precision=lax.Precision.HIGHEST,
                         preferred_element_type=jnp.float32)
    st += lax.dot_general(den, wct_ref[de:, :], (((1,), (0,)), ((), ())),
                          precision=lax.Precision.HIGHEST,
                          preferred_element_type=jnp.float32)
    st_ref[...] = st

    @pl.when(i == 0)
    def _():
        stats_e_ref[...] = jnp.zeros_like(stats_e_ref)
        stats_d_ref[...] = jnp.zeros_like(stats_d_ref)

    stats_e_ref[0:2, :] += jnp.concatenate(
        [jnp.sum(emb, axis=0, keepdims=True),
         jnp.sum(emb * emb, axis=0, keepdims=True)], axis=0)
    stats_d_ref[0:2, :] += jnp.concatenate(
        [jnp.sum(den, axis=0, keepdims=True),
         jnp.sum(den * den, axis=0, keepdims=True)], axis=0)


def _norm(x, stats_ref, g_ref, b_ref, inv_b):
    m = stats_ref[0:1, :] * inv_b
    v = stats_ref[1:2, :] * inv_b - m * m
    a = lax.rsqrt(v + _EPS) * g_ref[...]
    return (x - m) * a + b_ref[...]


def _pass_b(emb_ref, den_ref, stats_e_ref, stats_d_ref, ge_ref, be_ref,
            gd_ref, bd_ref, w1t_ref, b1_ref, h1_ref, stats1_ref, *, bsz, de):
    i = pl.program_id(0)
    inv_b = 1.0 / bsz
    xe = _norm(emb_ref[...], stats_e_ref, ge_ref, be_ref, inv_b)
    xd = _norm(den_ref[...], stats_d_ref, gd_ref, bd_ref, inv_b)
    h1 = lax.dot_general(xe, w1t_ref[0:de, :], (((1,), (0,)), ((), ())),
                         precision=lax.Precision.HIGHEST,
                         preferred_element_type=jnp.float32)
    h1 += lax.dot_general(xd, w1t_ref[de:, :], (((1,), (0,)), ((), ())),
                          precision=lax.Precision.HIGHEST,
                          preferred_element_type=jnp.float32)
    h1 += b1_ref[...]
    h1_ref[...] = h1

    @pl.when(i == 0)
    def _():
        stats1_ref[...] = jnp.zeros_like(stats1_ref)

    stats1_ref[0:2, :] += jnp.concatenate(
        [jnp.sum(h1, axis=0, keepdims=True),
         jnp.sum(h1 * h1, axis=0, keepdims=True)], axis=0)


def _pass_c(h1_ref, stats1_ref, g1_ref, b1n_ref, w2t_ref, b2_ref,
            h2_ref, stats2_ref, *, bsz):
    i = pl.program_id(0)
    xn = _norm(h1_ref[...], stats1_ref, g1_ref, b1n_ref, 1.0 / bsz)
    h2 = lax.dot_general(xn, w2t_ref[...], (((1,), (0,)), ((), ())),
                         precision=lax.Precision.HIGHEST,
                         preferred_element_type=jnp.float32) + b2_ref[...]
    h2_ref[...] = h2

    @pl.when(i == 0)
    def _():
        stats2_ref[...] = jnp.zeros_like(stats2_ref)

    stats2_ref[0:2, :] += jnp.concatenate(
        [jnp.sum(h2, axis=0, keepdims=True),
         jnp.sum(h2 * h2, axis=0, keepdims=True)], axis=0)


def _pass_d(h2_ref, st_ref, stats2_ref, g2_ref, b2n_ref, wpe_ref, wpd_ref,
            cb_ref, bp_ref, out_ref, *, bsz):
    h2n = _norm(h2_ref[...], stats2_ref, g2_ref, b2n_ref, 1.0 / bsz)
    dpart = jnp.sum(h2n * wpd_ref[...], axis=1, keepdims=True)
    wpe_sum = jnp.sum(wpe_ref[...], axis=1, keepdims=True)
    s = st_ref[:, 0:1]
    t = st_ref[:, 1:2]
    cb_last = cb_ref[0:1, _CROSS_L - 1:_CROSS_L]
    logit = (1.0 + s) * t + cb_last * wpe_sum + dpart + bp_ref[0:1, 0:1]
    out_ref[...] = jax.nn.sigmoid(logit)


def kernel(sparse_data, dense_data, emb_tables, cross_w, cross_b, bn0_g, bn0_b,
           w1, b1, bn1_g, bn1_b, w2, b2, bn2_g, bn2_b, wp, bp):
    bsz = sparse_data.shape[0]
    f, vocab, e = emb_tables.shape
    de = f * e
    dd = dense_data.shape[1]
    d_in = de + dd
    h1dim = w1.shape[0]
    h2dim = w2.shape[0]

    idx0 = sparse_data[:, 0].astype(jnp.int32)

    info = plsc.get_sparse_core_info()
    nw = info.num_cores * info.num_subcores
    assert bsz % (nw * _CHUNK) == 0
    nb_per_w = bsz // (nw * _CHUNK)
    assert (nb_per_w * f) % 2 == 0
    idx_w = idx0.reshape(nw, nb_per_w, _CHUNK)
    emb = _make_sc_gather(bsz, f, e, nb_per_w, info.num_cores)(
        emb_tables, idx_w)
    emb = emb.reshape(bsz, de)

    nblk = bsz // _BLK
    grid = (nblk,)
    row2 = lambda n: jnp.reshape(n, (1, -1))

    wct = jnp.stack([cross_w[_CROSS_L - 1], wp[0, :d_in]], axis=1)
    w1t = w1.T
    w2t = w2.T

    def bspec(shape_rows, shape_cols, blocked=True):
        if blocked:
            return pl.BlockSpec((shape_rows, shape_cols), lambda i: (i, 0))
        return pl.BlockSpec((shape_rows, shape_cols), lambda i: (0, 0))

    f32 = jnp.float32
    stats_e, stats_d, st = pl.pallas_call(
        functools.partial(_pass_a, de=de),
        grid=grid,
        in_specs=[
            bspec(_BLK, de), bspec(_BLK, dd), bspec(d_in, 2, False),
        ],
        out_specs=[
            bspec(8, de, False), bspec(8, dd, False), bspec(_BLK, 2),
        ],
        out_shape=[
            jax.ShapeDtypeStruct((8, de), f32),
            jax.ShapeDtypeStruct((8, dd), f32),
            jax.ShapeDtypeStruct((bsz, 2), f32),
        ],
    )(emb, dense_data, wct)

    h1, stats1 = pl.pallas_call(
        functools.partial(_pass_b, bsz=bsz, de=de),
        grid=grid,
        in_specs=[
            bspec(_BLK, de), bspec(_BLK, dd),
            bspec(8, de, False), bspec(8, dd, False),
            bspec(1, de, False), bspec(1, de, False),
            bspec(1, dd, False), bspec(1, dd, False),
            bspec(d_in, h1dim, False), bspec(1, h1dim, False),
        ],
        out_specs=[bspec(_BLK, h1dim), bspec(8, h1dim, False)],
        out_shape=[
            jax.ShapeDtypeStruct((bsz, h1dim), f32),
            jax.ShapeDtypeStruct((8, h1dim), f32),
        ],
    )(emb, dense_data, stats_e, stats_d,
      row2(bn0_g[:de]), row2(bn0_b[:de]), row2(bn0_g[de:]), row2(bn0_b[de:]),
      w1t, row2(b1))

    h2, stats2 = pl.pallas_call(
        functools.partial(_pass_c, bsz=bsz),
        grid=grid,
        in_specs=[
            bspec(_BLK, h1dim),
            bspec(8, h1dim, False),
            bspec(1, h1dim, False), bspec(1, h1dim, False),
            bspec(h1dim, h2dim, False), bspec(1, h2dim, False),
        ],
        out_specs=[bspec(_BLK, h2dim), bspec(8, h2dim, False)],
        out_shape=[
            jax.ShapeDtypeStruct((bsz, h2dim), f32),
            jax.ShapeDtypeStruct((8, h2dim), f32),
        ],
    )(h1, stats1, row2(bn1_g), row2(bn1_b), w2t, row2(b2))

    out = pl.pallas_call(
        functools.partial(_pass_d, bsz=bsz),
        grid=grid,
        in_specs=[
            bspec(_BLK, h2dim), bspec(_BLK, 2),
            bspec(8, h2dim, False),
            bspec(1, h2dim, False), bspec(1, h2dim, False),
            bspec(1, d_in, False), bspec(1, h2dim, False),
            bspec(1, _CROSS_L, False), bspec(1, 1, False),
        ],
        out_specs=bspec(_BLK, 1),
        out_shape=jax.ShapeDtypeStruct((bsz, 1), f32),
    )(h2, st, stats2, row2(bn2_g), row2(bn2_b),
      row2(wp[0, :d_in]), row2(wp[0, d_in:]), row2(cross_b), row2(bp))

    return out


# speedup vs baseline: 1.8027x; 1.1874x over previous
import functools

import jax
import jax.numpy as jnp
from jax import lax
from jax.experimental import pallas as pl
from jax.experimental.pallas import tpu as pltpu
from jax.experimental.pallas import tpu_sc as plsc

_CROSS_L = 3
_EPS = 1e-5
_BLK = 1024


_CHUNK = 128


def _make_sc_gather(bsz, f, e, nb_per_w, num_cores):
    mesh = plsc.VectorSubcoreMesh(core_axis_name="c", subcore_axis_name="s")
    nch_per_w = nb_per_w * f

    @functools.partial(
        pl.kernel,
        out_type=jax.ShapeDtypeStruct((bsz, f * e), jnp.float32),
        mesh=mesh,
        compiler_params=pltpu.CompilerParams(use_tc_tiling_on_sc=False),
        scratch_types=[
            pltpu.VMEM((nb_per_w, _CHUNK), jnp.int32),
            pltpu.VMEM((_CHUNK, e), jnp.float32),
            pltpu.VMEM((_CHUNK, e), jnp.float32),
            pltpu.SemaphoreType.DMA,
            pltpu.SemaphoreType.DMA,
        ],
    )
    def gather_k(table_hbm, idx_hbm, out_hbm, idx_v, buf0, buf1, sem0, sem1):
        wid = lax.axis_index("s") * num_cores + lax.axis_index("c")
        base = wid * nb_per_w * _CHUNK
        pltpu.sync_copy(idx_hbm.at[wid], idx_v)

        def src(c):
            r = c // f
            fld = c - r * f
            return table_hbm.at[fld].at[idx_v.at[r]]

        def start(c, buf, sem):
            pltpu.async_copy(src(c), buf, sem)

        def drain(c, buf, sem):
            r = c // f
            fld = c - r * f
            pltpu.make_async_copy(src(c), buf, sem).wait()
            pltpu.sync_copy(
                buf,
                out_hbm.at[pl.ds(base + r * _CHUNK, _CHUNK),
                           pl.ds(fld * e, e)])

        start(0, buf0, sem0)

        def body(i, carry):
            c = 2 * i
            start(c + 1, buf1, sem1)
            drain(c, buf0, sem0)

            @pl.when(c + 2 < nch_per_w)
            def _():
                start(c + 2, buf0, sem0)

            drain(c + 1, buf1, sem1)
            return carry

        lax.fori_loop(0, nch_per_w // 2, body, 0)

    return gather_k


def _pass_a(emb_ref, den_ref, wct_ref, stats_e_ref, stats_d_ref, st_ref, *, de):
    i = pl.program_id(0)
    emb = emb_ref[...]
    den = den_ref[...]
    st = lax.dot_general(emb, wct_ref[0:de, :], (((1,), (0,)), ((), ())),
                         precision=lax.Precision.HIGHEST,
                         preferred_element_type=jnp.float32)
    st += lax.dot_general(den, wct_ref[de:, :], (((1,), (0,)), ((), ())),
                          precision=lax.Precision.HIGHEST,
                          preferred_element_type=jnp.float32)
    st_ref[...] = st

    @pl.when(i == 0)
    def _():
        stats_e_ref[...] = jnp.zeros_like(stats_e_ref)
        stats_d_ref[...] = jnp.zeros_like(stats_d_ref)

    stats_e_ref[0:2, :] += jnp.concatenate(
        [jnp.sum(emb, axis=0, keepdims=True),
         jnp.sum(emb * emb, axis=0, keepdims=True)], axis=0)
    stats_d_ref[0:2, :] += jnp.concatenate(
        [jnp.sum(den, axis=0, keepdims=True),
         jnp.sum(den * den, axis=0, keepdims=True)], axis=0)


def _norm(x, stats_ref, g_ref, b_ref, inv_b):
    m = stats_ref[0:1, :] * inv_b
    v = stats_ref[1:2, :] * inv_b - m * m
    a = lax.rsqrt(v + _EPS) * g_ref[...]
    return (x - m) * a + b_ref[...]


def _pass_b(emb_ref, den_ref, stats_e_ref, stats_d_ref, ge_ref, be_ref,
            gd_ref, bd_ref, w1t_ref, b1_ref, h1_ref, stats1_ref, *, bsz, de):
    i = pl.program_id(0)
    inv_b = 1.0 / bsz
    xe = _norm(emb_ref[...], stats_e_ref, ge_ref, be_ref, inv_b)
    xd = _norm(den_ref[...], stats_d_ref, gd_ref, bd_ref, inv_b)
    h1 = lax.dot_general(xe, w1t_ref[0:de, :], (((1,), (0,)), ((), ())),
                         precision=lax.Precision.HIGHEST,
                         preferred_element_type=jnp.float32)
    h1 += lax.dot_general(xd, w1t_ref[de:, :], (((1,), (0,)), ((), ())),
                          precision=lax.Precision.HIGHEST,
                          preferred_element_type=jnp.float32)
    h1 += b1_ref[...]
    h1_ref[...] = h1

    @pl.when(i == 0)
    def _():
        stats1_ref[...] = jnp.zeros_like(stats1_ref)

    stats1_ref[0:2, :] += jnp.concatenate(
        [jnp.sum(h1, axis=0, keepdims=True),
         jnp.sum(h1 * h1, axis=0, keepdims=True)], axis=0)


def _pass_c(h1_ref, stats1_ref, g1_ref, b1n_ref, w2t_ref, b2_ref,
            h2_ref, stats2_ref, *, bsz):
    i = pl.program_id(0)
    xn = _norm(h1_ref[...], stats1_ref, g1_ref, b1n_ref, 1.0 / bsz)
    h2 = lax.dot_general(xn, w2t_ref[...], (((1,), (0,)), ((), ())),
                         precision=lax.Precision.HIGHEST,
                         preferred_element_type=jnp.float32) + b2_ref[...]
    h2_ref[...] = h2

    @pl.when(i == 0)
    def _():
        stats2_ref[...] = jnp.zeros_like(stats2_ref)

    stats2_ref[0:2, :] += jnp.concatenate(
        [jnp.sum(h2, axis=0, keepdims=True),
         jnp.sum(h2 * h2, axis=0, keepdims=True)], axis=0)


def _pass_d(h2_ref, st_ref, stats2_ref, g2_ref, b2n_ref, wpe_ref, wpd_ref,
            cb_ref, bp_ref, out_ref, *, bsz):
    h2n = _norm(h2_ref[...], stats2_ref, g2_ref, b2n_ref, 1.0 / bsz)
    dpart = jnp.sum(h2n * wpd_ref[...], axis=1, keepdims=True)
    wpe_sum = jnp.sum(wpe_ref[...], axis=1, keepdims=True)
    s = st_ref[:, 0:1]
    t = st_ref[:, 1:2]
    cb_last = cb_ref[0:1, _CROSS_L - 1:_CROSS_L]
    logit = (1.0 + s) * t + cb_last * wpe_sum + dpart + bp_ref[0:1, 0:1]
    out_ref[...] = jax.nn.sigmoid(logit)


def kernel(sparse_data, dense_data, emb_tables, cross_w, cross_b, bn0_g, bn0_b,
           w1, b1, bn1_g, bn1_b, w2, b2, bn2_g, bn2_b, wp, bp):
    bsz = sparse_data.shape[0]
    f, vocab, e = emb_tables.shape
    de = f * e
    dd = dense_data.shape[1]
    d_in = de + dd
    h1dim = w1.shape[0]
    h2dim = w2.shape[0]

    idx0 = sparse_data[:, 0].astype(jnp.int32)

    info = plsc.get_sparse_core_info()
    nw = info.num_cores * info.num_subcores
    assert bsz % (nw * _CHUNK) == 0
    nb_per_w = bsz // (nw * _CHUNK)
    assert (nb_per_w * f) % 2 == 0
    idx_w = idx0.reshape(nw, nb_per_w, _CHUNK)
    emb = _make_sc_gather(bsz, f, e, nb_per_w, info.num_cores)(
        emb_tables, idx_w)

    nblk = bsz // _BLK
    grid = (nblk,)
    row2 = lambda n: jnp.reshape(n, (1, -1))

    wct = jnp.stack([cross_w[_CROSS_L - 1], wp[0, :d_in]], axis=1)
    w1t = w1.T
    w2t = w2.T

    def bspec(shape_rows, shape_cols, blocked=True):
        if blocked:
            return pl.BlockSpec((shape_rows, shape_cols), lambda i: (i, 0))
        return pl.BlockSpec((shape_rows, shape_cols), lambda i: (0, 0))

    f32 = jnp.float32
    stats_e, stats_d, st = pl.pallas_call(
        functools.partial(_pass_a, de=de),
        grid=grid,
        in_specs=[
            bspec(_BLK, de), bspec(_BLK, dd), bspec(d_in, 2, False),
        ],
        out_specs=[
            bspec(8, de, False), bspec(8, dd, False), bspec(_BLK, 2),
        ],
        out_shape=[
            jax.ShapeDtypeStruct((8, de), f32),
            jax.ShapeDtypeStruct((8, dd), f32),
            jax.ShapeDtypeStruct((bsz, 2), f32),
        ],
    )(emb, dense_data, wct)

    h1, stats1 = pl.pallas_call(
        functools.partial(_pass_b, bsz=bsz, de=de),
        grid=grid,
        in_specs=[
            bspec(_BLK, de), bspec(_BLK, dd),
            bspec(8, de, False), bspec(8, dd, False),
            bspec(1, de, False), bspec(1, de, False),
            bspec(1, dd, False), bspec(1, dd, False),
            bspec(d_in, h1dim, False), bspec(1, h1dim, False),
        ],
        out_specs=[bspec(_BLK, h1dim), bspec(8, h1dim, False)],
        out_shape=[
            jax.ShapeDtypeStruct((bsz, h1dim), f32),
            jax.ShapeDtypeStruct((8, h1dim), f32),
        ],
    )(emb, dense_data, stats_e, stats_d,
      row2(bn0_g[:de]), row2(bn0_b[:de]), row2(bn0_g[de:]), row2(bn0_b[de:]),
      w1t, row2(b1))

    h2, stats2 = pl.pallas_call(
        functools.partial(_pass_c, bsz=bsz),
        grid=grid,
        in_specs=[
            bspec(_BLK, h1dim),
            bspec(8, h1dim, False),
            bspec(1, h1dim, False), bspec(1, h1dim, False),
            bspec(h1dim, h2dim, False), bspec(1, h2dim, False),
        ],
        out_specs=[bspec(_BLK, h2dim), bspec(8, h2dim, False)],
        out_shape=[
            jax.ShapeDtypeStruct((bsz, h2dim), f32),
            jax.ShapeDtypeStruct((8, h2dim), f32),
        ],
    )(h1, stats1, row2(bn1_g), row2(bn1_b), w2t, row2(b2))

    out = pl.pallas_call(
        functools.partial(_pass_d, bsz=bsz),
        grid=grid,
        in_specs=[
            bspec(_BLK, h2dim), bspec(_BLK, 2),
            bspec(8, h2dim, False),
            bspec(1, h2dim, False), bspec(1, h2dim, False),
            bspec(1, d_in, False), bspec(1, h2dim, False),
            bspec(1, _CROSS_L, False), bspec(1, 1, False),
        ],
        out_specs=bspec(_BLK, 1),
        out_shape=jax.ShapeDtypeStruct((bsz, 1), f32),
    )(h2, st, stats2, row2(bn2_g), row2(bn2_b),
      row2(wp[0, :d_in]), row2(wp[0, d_in:]), row2(cross_b), row2(bp))

    return out
